# baseline (speedup 1.0000x reference)
import jax
import jax.numpy as jnp
import numpy as np
from jax import lax
from jax.experimental import pallas as pl
from jax.experimental.pallas import tpu as pltpu
from jax.experimental.pallas import tpu_sc as plsc

B, L, F = 16, 4096, 16
NC, NS = 2, 16
NW = NC * NS
SCK = 2
NVR = L // 16

_MAGIC = np.float32(12582912.0)
_INV_PI = np.float32(0.3183098861837907)
_C1 = np.float32(3.141584873)
_C3 = np.float32(-5.167248249)
_C5 = np.float32(2.542875767)
_C7 = np.float32(-0.5571599603)


def _sin_pi(a):
    y = a + _MAGIC
    ib = lax.bitcast_convert_type(y, jnp.int32)
    sgn = (ib & 1) << 31
    nf = y - _MAGIC
    r = a - nf
    u = r * r
    p = ((_C7 * u + _C5) * u + _C3) * u + _C1
    sv = r * p
    return lax.bitcast_convert_type(
        lax.bitcast_convert_type(sv, jnp.int32) ^ sgn, jnp.float32)


def _tanh_exp(x):
    e = jnp.exp(x + x)
    return 1.0 - 2.0 / (e + 1.0)


def _lanesum(v, lane):
    for sh in (1, 2, 4, 8):
        v = v + v[lane ^ sh]
    return v


def _sc_body(ts_hbm, om_hbm, ph_hbm, out_hbm, part_hbm,
             ts_v, om16_v, ph16_v, pub_v, pe_v, out_v):
    c = lax.axis_index("c")
    s = lax.axis_index("s")
    lane = lax.iota(jnp.int32, 16)

    pltpu.sync_copy(ts_hbm.at[c], ts_v)
    pltpu.sync_copy(om_hbm, om16_v)
    pltpu.sync_copy(ph_hbm, ph16_v)

    o_raw = om16_v[...]
    p_raw = ph16_v[...]
    o_pi = o_raw * _INV_PI
    p_pi = p_raw * _INV_PI
    kidx = jnp.full((16,), s, jnp.int32)
    ok_raw = o_raw[kidx]
    pk_raw = p_raw[kidx]
    ok_pi = o_pi[kidx]
    pk_pi = p_pi[kidx]
    is_lin = s == 0

    zeros = jnp.zeros((16,), jnp.float32)

    @plsc.parallel_loop(0, NVR, 1, unroll=4, carry=(zeros, zeros))
    def acc(i, carry):
        n_acc, d_acc = carry
        v = ts_v[pl.ds(i * 16, 16)]
        mf = jnp.where(v != 0.0, 1.0, 0.0)
        w = jnp.exp(v) * mf
        f = jnp.where(is_lin, ok_raw * v + pk_raw, _sin_pi(ok_pi * v + pk_pi))
        return n_acc + w * f, d_acc + w

    n_acc, d_acc = acc
    pub_v[0] = _lanesum(n_acc, lane)
    pub_v[1] = _lanesum(d_acc, lane)
    pltpu.sync_copy(pub_v, part_hbm.at[pl.ds(2 * (NS * c + s), 2)])
    plsc.subcore_barrier()

    @pl.when(s == 0)
    def _epilogue():
        pltpu.sync_copy(part_hbm.at[pl.ds(2 * NS * c, 2 * NS)], pe_v)
        d_s = pe_v[1]
        n_vec = jnp.zeros((16,), jnp.float32)
        for k in range(F):
            n_vec = n_vec + jnp.where(lane == k, pe_v[2 * k], 0.0)
        out_v[...] = _tanh_exp(n_vec / (d_s + 1e-8))
        pltpu.sync_copy(out_v, out_hbm.at[pl.ds(c * F, F)])


def _tc_body(om_ref, ph_ref, ts_ref, out_ref):
    t = ts_ref[...]
    mf = jnp.where(t != 0.0, 1.0, 0.0)
    w = jnp.exp(t) * mf
    d = jnp.sum(w, axis=1, keepdims=True)
    cols = []
    for k in range(F):
        om = om_ref[k]
        phv = ph_ref[k]
        f = om * t + phv if k == 0 else jnp.sin(om * t + phv)
        cols.append(jnp.sum(w * f, axis=1, keepdims=True))
    n = jnp.concatenate(cols, axis=1)
    out_ref[...] = jnp.tanh(n / (d + 1e-8))


@jax.jit
def kernel(timestamps, omega, phi):
    mesh = plsc.VectorSubcoreMesh(core_axis_name="c", subcore_axis_name="s")
    sc_run = pl.kernel(
        _sc_body,
        mesh=mesh,
        out_type=(
            jax.ShapeDtypeStruct((SCK * F,), jnp.float32),
            jax.ShapeDtypeStruct((2 * NW, 16), jnp.float32),
        ),
        scratch_types=[
            pltpu.VMEM((L,), jnp.float32),
            pltpu.VMEM((F,), jnp.float32),
            pltpu.VMEM((F,), jnp.float32),
            pltpu.VMEM((2, 16), jnp.float32),
            pltpu.VMEM((2 * NS, 16), jnp.float32),
            pltpu.VMEM((16,), jnp.float32),
        ],
    )
    out_sc, _ = sc_run(timestamps[:SCK], omega, phi)

    out_tc = pl.pallas_call(
        _tc_body,
        out_shape=jax.ShapeDtypeStruct((B - SCK, F), jnp.float32),
        in_specs=[
            pl.BlockSpec(memory_space=pltpu.SMEM),
            pl.BlockSpec(memory_space=pltpu.SMEM),
            pl.BlockSpec((B - SCK, L), lambda: (0, 0)),
        ],
        out_specs=pl.BlockSpec((B - SCK, F), lambda: (0, 0)),
    )(omega, phi, timestamps[SCK:])

    return jnp.concatenate([out_sc.reshape(SCK, F), out_tc], axis=0)


# speedup vs baseline: 1.1983x; 1.1983x over previous
import jax
import jax.numpy as jnp
import numpy as np
from jax import lax
from jax.experimental import pallas as pl
from jax.experimental.pallas import tpu as pltpu
from jax.experimental.pallas import tpu_sc as plsc

B, L, F = 16, 4096, 16
NC, NS = 2, 16
NW = NC * NS
SCK = 8
ROWS_PER_CORE = SCK // NC
FPG = F // 4
NVR = L // 16

_MAGIC = np.float32(12582912.0)
_INV_PI = np.float32(0.3183098861837907)
_C1 = np.float32(3.141584873)
_C3 = np.float32(-5.167248249)
_C5 = np.float32(2.542875767)
_C7 = np.float32(-0.5571599603)


def _sin_pi(a):
    y = a + _MAGIC
    ib = lax.bitcast_convert_type(y, jnp.int32)
    sgn = (ib & 1) << 31
    nf = y - _MAGIC
    r = a - nf
    u = r * r
    p = ((_C7 * u + _C5) * u + _C3) * u + _C1
    sv = r * p
    return lax.bitcast_convert_type(
        lax.bitcast_convert_type(sv, jnp.int32) ^ sgn, jnp.float32)


def _tanh_exp(x):
    e = jnp.exp(x + x)
    return 1.0 - 2.0 / (e + 1.0)


def _lanesum(v, lane):
    for sh in (1, 2, 4, 8):
        v = v + v[lane ^ sh]
    return v


def _sc_body(ts_hbm, om_hbm, ph_hbm, out_hbm, part_hbm,
             ts_v, w_v, om16_v, ph16_v, pub_v, pe_v, out_v):
    c = lax.axis_index("c")
    s = lax.axis_index("s")
    lane = lax.iota(jnp.int32, 16)
    g = s % 4
    row = ROWS_PER_CORE * c + s // 4

    pltpu.sync_copy(ts_hbm.at[row], ts_v)
    pltpu.sync_copy(om_hbm, om16_v)
    pltpu.sync_copy(ph_hbm, ph16_v)

    o_raw = om16_v[...]
    p_raw = ph16_v[...]
    o_pi = o_raw * _INV_PI
    p_pi = p_raw * _INV_PI

    zeros = jnp.zeros((16,), jnp.float32)

    @plsc.parallel_loop(0, NVR, 1, unroll=8, carry=zeros)
    def pass_a(i, d_acc):
        v = ts_v[pl.ds(i * 16, 16)]
        mf = jnp.where(v != 0.0, 1.0, 0.0)
        w = jnp.exp(v) * mf
        w_v[pl.ds(i * 16, 16)] = w
        return d_acc + w

    d_acc = pass_a

    for j in range(FPG):
        kf = 4 * g + j
        kidx = jnp.full((16,), kf, jnp.int32)
        ok_pi = o_pi[kidx]
        pk_pi = p_pi[kidx]
        if j == 0:
            ok_raw = o_raw[kidx]
            pk_raw = p_raw[kidx]
            is_lin = kf == 0

            @plsc.parallel_loop(0, NVR, 1, unroll=8, carry=zeros)
            def pass_b(i, n_acc):
                v = ts_v[pl.ds(i * 16, 16)]
                w = w_v[pl.ds(i * 16, 16)]
                f = jnp.where(is_lin, ok_raw * v + pk_raw,
                              _sin_pi(ok_pi * v + pk_pi))
                return n_acc + w * f
        else:
            @plsc.parallel_loop(0, NVR, 1, unroll=8, carry=zeros)
            def pass_b(i, n_acc):
                v = ts_v[pl.ds(i * 16, 16)]
                w = w_v[pl.ds(i * 16, 16)]
                return n_acc + w * _sin_pi(ok_pi * v + pk_pi)

        pub_v[j] = _lanesum(pass_b, lane)

    pub_v[FPG] = _lanesum(d_acc, lane)
    pltpu.sync_copy(pub_v, part_hbm.at[NS * c + s])
    plsc.subcore_barrier()

    @pl.when(g == 0)
    def _epilogue():
        pltpu.sync_copy(part_hbm.at[pl.ds(NS * c + s, 4)], pe_v)
        d_s = pe_v[0, FPG]
        n_vec = jnp.zeros((16,), jnp.float32)
        for j in range(4):
            for i in range(FPG):
                n_vec = n_vec + jnp.where(lane == 4 * j + i,
                                          pe_v[j, i], 0.0)
        out_v[...] = _tanh_exp(n_vec / (d_s + 1e-8))
        pltpu.sync_copy(out_v, out_hbm.at[pl.ds(row * F, F)])


def _tc_body(om_ref, ph_ref, ts_ref, out_ref):
    t = ts_ref[...]
    mf = jnp.where(t != 0.0, 1.0, 0.0)
    w = jnp.exp(t) * mf
    d = jnp.sum(w, axis=1, keepdims=True)
    cols = [jnp.sum(w * (om_ref[0] * t + ph_ref[0]), axis=1, keepdims=True)]
    for k in range(1, F):
        ok = om_ref[k] * _INV_PI
        pk = ph_ref[k] * _INV_PI
        cols.append(
            jnp.sum(w * _sin_pi(ok * t + pk), axis=1, keepdims=True))
    n = jnp.concatenate(cols, axis=1)
    out_ref[...] = jnp.tanh(n / (d + 1e-8))


@jax.jit
def kernel(timestamps, omega, phi):
    mesh = plsc.VectorSubcoreMesh(core_axis_name="c", subcore_axis_name="s")
    sc_run = pl.kernel(
        _sc_body,
        mesh=mesh,
        out_type=(
            jax.ShapeDtypeStruct((SCK * F,), jnp.float32),
            jax.ShapeDtypeStruct((NW, FPG + 1, 16), jnp.float32),
        ),
        scratch_types=[
            pltpu.VMEM((L,), jnp.float32),
            pltpu.VMEM((L,), jnp.float32),
            pltpu.VMEM((F,), jnp.float32),
            pltpu.VMEM((F,), jnp.float32),
            pltpu.VMEM((FPG + 1, 16), jnp.float32),
            pltpu.VMEM((4, FPG + 1, 16), jnp.float32),
            pltpu.VMEM((16,), jnp.float32),
        ],
    )
    out_sc, _ = sc_run(timestamps, omega, phi)

    out_tc = pl.pallas_call(
        _tc_body,
        grid=(1,),
        out_shape=jax.ShapeDtypeStruct((B, F), jnp.float32),
        in_specs=[
            pl.BlockSpec(memory_space=pltpu.SMEM),
            pl.BlockSpec(memory_space=pltpu.SMEM),
            pl.BlockSpec((SCK, L), lambda j: (1, 0)),
        ],
        out_specs=pl.BlockSpec((SCK, F), lambda j: (1, 0)),
    )(omega, phi, timestamps)

    return lax.dynamic_update_slice(out_tc, out_sc.reshape(SCK, F), (0, 0))


# speedup vs baseline: 1.3130x; 1.0958x over previous
import jax
import jax.numpy as jnp
import numpy as np
from jax import lax
from jax.experimental import pallas as pl
from jax.experimental.pallas import tpu as pltpu
from jax.experimental.pallas import tpu_sc as plsc

B, L, F = 16, 4096, 16
NC, NS = 2, 16
NW = NC * NS
SCK = 2
NVR = L // 16

_MAGIC = np.float32(12582912.0)
_INV_PI = np.float32(0.3183098861837907)
_C1 = np.float32(3.141584873)
_C3 = np.float32(-5.167248249)
_C5 = np.float32(2.542875767)
_C7 = np.float32(-0.5571599603)


def _sin_pi(a):
    y = a + _MAGIC
    ib = lax.bitcast_convert_type(y, jnp.int32)
    sgn = (ib & 1) << 31
    nf = y - _MAGIC
    r = a - nf
    u = r * r
    p = ((_C7 * u + _C5) * u + _C3) * u + _C1
    sv = r * p
    return lax.bitcast_convert_type(
        lax.bitcast_convert_type(sv, jnp.int32) ^ sgn, jnp.float32)


def _tanh_exp(x):
    e = jnp.exp(x + x)
    return 1.0 - 2.0 / (e + 1.0)


def _lanesum(v, lane):
    for sh in (1, 2, 4, 8):
        v = v + v[lane ^ sh]
    return v


def _sc_body(ts_hbm, om_hbm, ph_hbm, out_hbm, part_hbm,
             ts_v, w_v, om16_v, ph16_v, pub_v, pe_v, out_v):
    c = lax.axis_index("c")
    s = lax.axis_index("s")
    lane = lax.iota(jnp.int32, 16)

    pltpu.sync_copy(ts_hbm.at[c], ts_v)
    pltpu.sync_copy(om_hbm, om16_v)
    pltpu.sync_copy(ph_hbm, ph16_v)

    o_raw = om16_v[...]
    p_raw = ph16_v[...]
    o_pi = o_raw * _INV_PI
    p_pi = p_raw * _INV_PI
    kidx = jnp.full((16,), s, jnp.int32)
    ok_raw = o_raw[kidx]
    pk_raw = p_raw[kidx]
    ok_pi = o_pi[kidx]
    pk_pi = p_pi[kidx]
    is_lin = s == 0

    zeros = jnp.zeros((16,), jnp.float32)

    @plsc.parallel_loop(0, NVR, 1, unroll=8, carry=zeros)
    def pass_a(i, d_acc):
        v = ts_v[pl.ds(i * 16, 16)]
        mf = jnp.where(v != 0.0, 1.0, 0.0)
        w = jnp.exp(v) * mf
        w_v[pl.ds(i * 16, 16)] = w
        return d_acc + w

    d_acc = pass_a

    @plsc.parallel_loop(0, NVR, 1, unroll=8, carry=zeros)
    def pass_b(i, n_acc):
        v = ts_v[pl.ds(i * 16, 16)]
        w = w_v[pl.ds(i * 16, 16)]
        f = jnp.where(is_lin, ok_raw * v + pk_raw,
                      _sin_pi(ok_pi * v + pk_pi))
        return n_acc + w * f

    pub_v[0] = _lanesum(pass_b, lane)
    pub_v[1] = _lanesum(d_acc, lane)
    pltpu.sync_copy(pub_v, part_hbm.at[NS * c + s])
    plsc.subcore_barrier()

    @pl.when(s == 0)
    def _epilogue():
        pltpu.sync_copy(part_hbm.at[pl.ds(NS * c, NS)], pe_v)
        d_s = pe_v[0, 1]
        n_vec = jnp.zeros((16,), jnp.float32)
        for k in range(F):
            n_vec = n_vec + jnp.where(lane == k, pe_v[k, 0], 0.0)
        out_v[...] = _tanh_exp(n_vec / (d_s + 1e-8))
        pltpu.sync_copy(out_v, out_hbm.at[pl.ds(c * F, F)])


def _tc_body(om_ref, ph_ref, ts_ref, out_ref):
    t = ts_ref[...][SCK:, :]
    mf = jnp.where(t != 0.0, 1.0, 0.0)
    w = jnp.exp(t) * mf
    d = jnp.sum(w, axis=1, keepdims=True)
    cols = [jnp.sum(w * (om_ref[0] * t + ph_ref[0]), axis=1, keepdims=True)]
    for k in range(1, F):
        ok = om_ref[k] * _INV_PI
        pk = ph_ref[k] * _INV_PI
        cols.append(
            jnp.sum(w * _sin_pi(ok * t + pk), axis=1, keepdims=True))
    n = jnp.concatenate(cols, axis=1)
    out = jnp.tanh(n / (d + 1e-8))
    out_ref[...] = jnp.concatenate(
        [jnp.zeros((SCK, F), jnp.float32), out], axis=0)


@jax.jit
def kernel(timestamps, omega, phi):
    mesh = plsc.VectorSubcoreMesh(core_axis_name="c", subcore_axis_name="s")
    sc_run = pl.kernel(
        _sc_body,
        mesh=mesh,
        out_type=(
            jax.ShapeDtypeStruct((SCK * F,), jnp.float32),
            jax.ShapeDtypeStruct((NW, 2, 16), jnp.float32),
        ),
        scratch_types=[
            pltpu.VMEM((L,), jnp.float32),
            pltpu.VMEM((L,), jnp.float32),
            pltpu.VMEM((F,), jnp.float32),
            pltpu.VMEM((F,), jnp.float32),
            pltpu.VMEM((2, 16), jnp.float32),
            pltpu.VMEM((NS, 2, 16), jnp.float32),
            pltpu.VMEM((16,), jnp.float32),
        ],
    )
    out_sc, _ = sc_run(timestamps, omega, phi)

    out_tc = pl.pallas_call(
        _tc_body,
        out_shape=jax.ShapeDtypeStruct((B, F), jnp.float32),
        in_specs=[
            pl.BlockSpec(memory_space=pltpu.SMEM),
            pl.BlockSpec(memory_space=pltpu.SMEM),
            pl.BlockSpec((B, L), lambda: (0, 0)),
        ],
        out_specs=pl.BlockSpec((B, F), lambda: (0, 0)),
    )(omega, phi, timestamps)

    return lax.dynamic_update_slice(out_tc, out_sc.reshape(SCK, F), (0, 0))


# speedup vs baseline: 1.4276x; 1.0873x over previous
import jax
import jax.numpy as jnp
import numpy as np
from jax import lax
from jax.experimental import pallas as pl
from jax.experimental.pallas import tpu as pltpu
from jax.experimental.pallas import tpu_sc as plsc

B, L, F = 16, 4096, 16
NC, NS = 2, 16
NW = NC * NS
SCK = 2
NVR = L // 16

_MAGIC = np.float32(12582912.0)
_INV_PI = np.float32(0.3183098861837907)
_C1 = np.float32(3.141584873)
_C3 = np.float32(-5.167248249)
_C5 = np.float32(2.542875767)
_C7 = np.float32(-0.5571599603)


def _sin_pi(a):
    y = a + _MAGIC
    ib = lax.bitcast_convert_type(y, jnp.int32)
    sgn = (ib & 1) << 31
    nf = y - _MAGIC
    r = a - nf
    u = r * r
    p = ((_C7 * u + _C5) * u + _C3) * u + _C1
    sv = r * p
    return lax.bitcast_convert_type(
        lax.bitcast_convert_type(sv, jnp.int32) ^ sgn, jnp.float32)


def _tanh_exp(x):
    e = jnp.exp(x + x)
    return 1.0 - 2.0 / (e + 1.0)


def _lanesum(v, lane):
    for sh in (1, 2, 4, 8):
        v = v + v[lane ^ sh]
    return v


def _sc_body(ts_hbm, om_hbm, ph_hbm, part_hbm,
             ts_v, w_v, om16_v, ph16_v, pub_v, sem):
    c = lax.axis_index("c")
    s = lax.axis_index("s")
    lane = lax.iota(jnp.int32, 16)

    cp1 = pltpu.async_copy(ts_hbm.at[c], ts_v, sem)
    cp2 = pltpu.async_copy(om_hbm, om16_v, sem)
    cp3 = pltpu.async_copy(ph_hbm, ph16_v, sem)
    cp1.wait()
    cp2.wait()
    cp3.wait()

    o_raw = om16_v[...]
    p_raw = ph16_v[...]
    o_pi = o_raw * _INV_PI
    p_pi = p_raw * _INV_PI
    kidx = jnp.full((16,), s, jnp.int32)
    ok_raw = o_raw[kidx]
    pk_raw = p_raw[kidx]
    ok_pi = o_pi[kidx]
    pk_pi = p_pi[kidx]
    is_lin = s == 0

    zeros = jnp.zeros((16,), jnp.float32)

    @plsc.parallel_loop(0, NVR, 1, unroll=8, carry=zeros)
    def pass_a(i, d_acc):
        v = ts_v[pl.ds(i * 16, 16)]
        mf = jnp.where(v != 0.0, 1.0, 0.0)
        w = jnp.exp(v) * mf
        w_v[pl.ds(i * 16, 16)] = w
        return d_acc + w

    d_acc = pass_a

    @plsc.parallel_loop(0, NVR, 1, unroll=8, carry=zeros)
    def pass_b(i, n_acc):
        v = ts_v[pl.ds(i * 16, 16)]
        w = w_v[pl.ds(i * 16, 16)]
        f = jnp.where(is_lin, ok_raw * v + pk_raw,
                      _sin_pi(ok_pi * v + pk_pi))
        return n_acc + w * f

    n_s = _lanesum(pass_b, lane)
    d_s = _lanesum(d_acc, lane)
    val = _tanh_exp(n_s / (d_s + 1e-8))
    pub_v[0] = jnp.where(lane == s, val, 0.0)
    pltpu.sync_copy(pub_v, part_hbm.at[NS * c + s])


def _tc_body(om_ref, ph_ref, ts_ref, out_ref):
    t = ts_ref[...][SCK:, :]
    mf = jnp.where(t != 0.0, 1.0, 0.0)
    w = jnp.exp(t) * mf
    d = jnp.sum(w, axis=1, keepdims=True)
    cols = [jnp.sum(w * (om_ref[0] * t + ph_ref[0]), axis=1, keepdims=True)]
    for k in range(1, F):
        ok = om_ref[k] * _INV_PI
        pk = ph_ref[k] * _INV_PI
        cols.append(
            jnp.sum(w * _sin_pi(ok * t + pk), axis=1, keepdims=True))
    n = jnp.concatenate(cols, axis=1)
    out = jnp.tanh(n / (d + 1e-8))
    out_ref[...] = jnp.concatenate(
        [jnp.zeros((SCK, F), jnp.float32), out], axis=0)


@jax.jit
def kernel(timestamps, omega, phi):
    mesh = plsc.VectorSubcoreMesh(core_axis_name="c", subcore_axis_name="s")
    sc_run = pl.kernel(
        _sc_body,
        mesh=mesh,
        out_type=jax.ShapeDtypeStruct((NW, 1, 16), jnp.float32),
        scratch_types=[
            pltpu.VMEM((L,), jnp.float32),
            pltpu.VMEM((L,), jnp.float32),
            pltpu.VMEM((F,), jnp.float32),
            pltpu.VMEM((F,), jnp.float32),
            pltpu.VMEM((1, 16), jnp.float32),
            pltpu.SemaphoreType.DMA,
        ],
    )
    part = sc_run(timestamps, omega, phi)
    out_sc = jnp.sum(part.reshape(SCK, NS, F), axis=1)

    out_tc = pl.pallas_call(
        _tc_body,
        out_shape=jax.ShapeDtypeStruct((B, F), jnp.float32),
        in_specs=[
            pl.BlockSpec(memory_space=pltpu.SMEM),
            pl.BlockSpec(memory_space=pltpu.SMEM),
            pl.BlockSpec((B, L), lambda: (0, 0)),
        ],
        out_specs=pl.BlockSpec((B, F), lambda: (0, 0)),
    )(omega, phi, timestamps)

    return lax.dynamic_update_slice(out_tc, out_sc, (0, 0))
